# baseline (speedup 1.0000x reference)
import functools

import jax
import jax.numpy as jnp
from jax import lax
from jax.experimental import pallas as pl
from jax.experimental.pallas import tpu as pltpu
from jax.experimental.pallas import tpu_sc as plsc

B_, S_, D_, H_, E_ = 2, 2048, 1024, 2048, 8
T_ = B_ * S_
C_ = 640
NSLOT = E_ * C_
NC, NS = 2, 16
NW = NC * NS
TPW = T_ // NW
RBUF = NSLOT + NW

TB = 512
HT = 512
HTW = 512
ESC = 4
RPW = ESC * H_ // NW
RCH = 16


def _w2sum_sc_body(w2f_hbm, out_hbm, rows_a, rows_b, sums_v, sem_a, sem_b):
    wid = lax.axis_index("c") * NS + lax.axis_index("s")
    rbase = wid * RPW

    def _process(rows_v, g):
        for r in range(RCH):
            acc = jnp.zeros((16,), jnp.float32)
            for j in range(D_ // 16):
                acc = acc + rows_v[r, pl.ds(j * 16, 16)]
            sums_v[g * RCH + r, :] = acc

    nch = RPW // RCH
    pltpu.async_copy(w2f_hbm.at[pl.ds(rbase, RCH)], rows_a, sem_a)

    def body(i, carry):
        g = i * 2
        pltpu.make_async_copy(w2f_hbm.at[pl.ds(rbase, RCH)], rows_a, sem_a).wait()
        pltpu.async_copy(
            w2f_hbm.at[pl.ds(rbase + (g + 1) * RCH, RCH)], rows_b, sem_b)
        _process(rows_a, g)
        pltpu.make_async_copy(w2f_hbm.at[pl.ds(rbase, RCH)], rows_b, sem_b).wait()

        @pl.when(g + 2 < nch)
        def _():
            pltpu.async_copy(
                w2f_hbm.at[pl.ds(rbase + (g + 2) * RCH, RCH)], rows_a, sem_a)

        _process(rows_b, g + 1)
        return carry

    lax.fori_loop(0, nch // 2, body, 0)
    pltpu.sync_copy(sums_v, out_hbm.at[pl.ds(rbase, RPW)])


def _w2sum_sc(w2flat):
    mesh = plsc.VectorSubcoreMesh(core_axis_name="c", subcore_axis_name="s")
    return pl.kernel(
        _w2sum_sc_body,
        out_type=jax.ShapeDtypeStruct((ESC * H_, 16), jnp.float32),
        mesh=mesh,
        scratch_types=[
            pltpu.VMEM((RCH, D_), jnp.float32),
            pltpu.VMEM((RCH, D_), jnp.float32),
            pltpu.VMEM((RPW, 16), jnp.float32),
            pltpu.SemaphoreType.DMA,
            pltpu.SemaphoreType.DMA,
        ],
    )(w2flat)


def _w2sum_tc_body(w2_ref, out_ref):
    out_ref[0] = jnp.sum(w2_ref[0], axis=1, keepdims=True)


def _w2sum_tc(w2):
    return pl.pallas_call(
        _w2sum_tc_body,
        grid=(E_ - ESC, H_ // HTW),
        in_specs=[pl.BlockSpec((1, HTW, D_), lambda e, h: (e + ESC, h, 0))],
        out_specs=pl.BlockSpec((1, HTW, 1), lambda e, h: (e, h, 0)),
        out_shape=jax.ShapeDtypeStruct((E_ - ESC, H_, 1), jnp.float32),
    )(w2)


def _gate_body(x_ref, wg_ref, slot_ref, gmul_ref, dst_ref, carry_ref):
    i = pl.program_id(0)
    x = x_ref[...]
    logits = jnp.dot(x, wg_ref[...], preferred_element_type=jnp.float32)
    lmax = jnp.max(logits, axis=1, keepdims=True)
    gval = 1.0 / jnp.sum(jnp.exp(logits - lmax), axis=1, keepdims=True)
    eids = lax.broadcasted_iota(jnp.int32, logits.shape, 1)
    eidx = jnp.min(jnp.where(logits >= lmax, eids, E_), axis=1, keepdims=True)

    @pl.when(i == 0)
    def _():
        carry_ref[...] = jnp.zeros_like(carry_ref)

    onehot = (eids == eidx).astype(jnp.float32)
    r = lax.broadcasted_iota(jnp.int32, (TB, TB), 0)
    c = lax.broadcasted_iota(jnp.int32, (TB, TB), 1)
    tri = (r >= c).astype(jnp.float32)
    cnt = jnp.dot(tri, onehot, preferred_element_type=jnp.float32)
    cnt = cnt + carry_ref[...]
    carry_ref[...] = cnt[TB - 1:TB, :]
    pos = jnp.sum(onehot * cnt, axis=1, keepdims=True).astype(jnp.int32) - 1
    valid = pos < C_
    slot = eidx * C_ + jnp.minimum(pos, C_ - 1)
    trow = lax.broadcasted_iota(jnp.int32, (TB, 1), 0) + i * TB
    dump = NSLOT + trow // TPW
    slot_ref[...] = slot.reshape(TB)
    gmul_ref[...] = jnp.where(valid, gval, 0.0).reshape(TB)
    dst_ref[...] = jnp.where(valid, slot, dump).reshape(TB)


def _gating(xf, wg):
    return pl.pallas_call(
        _gate_body,
        grid=(T_ // TB,),
        in_specs=[
            pl.BlockSpec((TB, D_), lambda i: (i, 0)),
            pl.BlockSpec((D_, E_), lambda i: (0, 0)),
        ],
        out_specs=[
            pl.BlockSpec((TB,), lambda i: (i,)),
            pl.BlockSpec((TB,), lambda i: (i,)),
            pl.BlockSpec((TB,), lambda i: (i,)),
        ],
        out_shape=[
            jax.ShapeDtypeStruct((T_,), jnp.int32),
            jax.ShapeDtypeStruct((T_,), jnp.float32),
            jax.ShapeDtypeStruct((T_,), jnp.int32),
        ],
        scratch_shapes=[pltpu.VMEM((1, E_), jnp.float32)],
    )(xf, wg)


def _scatter_body(dst_hbm, x_hbm, bufx_hbm, dst_v, rows_v, sem):
    wid = lax.axis_index("c") * NS + lax.axis_index("s")
    base = wid * TPW
    pltpu.sync_copy(dst_hbm.at[pl.ds(base, TPW)], dst_v)
    for v in range(TPW // 16):
        idx = dst_v[pl.ds(v * 16, 16)]
        pltpu.sync_copy(x_hbm.at[pl.ds(base + v * 16, 16)], rows_v)
        pltpu.async_copy(rows_v, bufx_hbm.at[idx], sem).wait()


def _scatter(dst, xf):
    mesh = plsc.VectorSubcoreMesh(core_axis_name="c", subcore_axis_name="s")
    return pl.kernel(
        _scatter_body,
        out_type=jax.ShapeDtypeStruct((RBUF, D_), jnp.float32),
        mesh=mesh,
        scratch_types=[
            pltpu.VMEM((TPW,), jnp.int32),
            pltpu.VMEM((16, D_), jnp.float32),
            pltpu.SemaphoreType.DMA,
        ],
    )(dst, xf)


def _ffn_body(bufx_ref, w1_ref, b1_ref, w2sa_ref, w2sb_ref, b2_ref,
              out_ref, acc_ref, xb_ref):
    e = pl.program_id(0)
    ht = pl.program_id(1)

    @pl.when(ht == 0)
    def _():
        xb_ref[...] = bufx_ref[...].astype(jnp.bfloat16)

    w1 = w1_ref[0].astype(jnp.bfloat16)
    hblk = jnp.dot(xb_ref[...], w1, preferred_element_type=jnp.float32)
    hblk = jnp.maximum(hblk + b1_ref[0], 0.0)
    w2sa = jnp.sum(w2sa_ref[...], axis=1, keepdims=True)
    w2s = jnp.where(e < ESC, w2sa, w2sb_ref[0])
    part = jnp.dot(hblk, w2s, preferred_element_type=jnp.float32)

    @pl.when(ht == 0)
    def _():
        acc_ref[...] = part + jnp.sum(b2_ref[0])

    @pl.when(ht != 0)
    def _():
        acc_ref[...] += part

    @pl.when(ht == H_ // HT - 1)
    def _():
        out_ref[pl.ds(e * C_, C_)] = acc_ref[...].reshape(C_)


def _ffn(bufx, w1, b1r, w2sa, w2sb, b2r):
    return pl.pallas_call(
        _ffn_body,
        grid=(E_, H_ // HT),
        in_specs=[
            pl.BlockSpec((C_, D_), lambda e, h: (e, 0)),
            pl.BlockSpec((1, D_, HT), lambda e, h: (e, 0, h)),
            pl.BlockSpec((1, 1, HT), lambda e, h: (e, 0, h)),
            pl.BlockSpec(
                (HT, 16),
                lambda e, h: (jnp.minimum(e, ESC - 1) * (H_ // HT) + h, 0)),
            pl.BlockSpec((1, HT, 1), lambda e, h: (jnp.maximum(e - ESC, 0), h, 0)),
            pl.BlockSpec((1, 1, D_), lambda e, h: (e, 0, 0)),
        ],
        out_specs=pl.BlockSpec((NSLOT,), lambda e, h: (0,)),
        out_shape=jax.ShapeDtypeStruct((NSLOT,), jnp.float32),
        scratch_shapes=[pltpu.VMEM((C_, 1), jnp.float32),
                        pltpu.VMEM((C_, D_), jnp.bfloat16)],
    )(bufx, w1, b1r, w2sa, w2sb, b2r)


def _combine_body(slot_hbm, gmul_hbm, s1_hbm, z_hbm, sl_v, gm_v, val_v, z_v, sem):
    wid = lax.axis_index("c") * NS + lax.axis_index("s")
    base = wid * TPW
    pltpu.sync_copy(slot_hbm.at[pl.ds(base, TPW)], sl_v)
    pltpu.sync_copy(gmul_hbm.at[pl.ds(base, TPW)], gm_v)
    for v in range(TPW // 16):
        idx = sl_v[pl.ds(v * 16, 16)]
        pltpu.async_copy(s1_hbm.at[idx], val_v, sem).wait()
        z_v[pl.ds(v * 16, 16)] = val_v[...] * gm_v[pl.ds(v * 16, 16)]
    row = wid // (S_ // TPW)
    col = (wid % (S_ // TPW)) * TPW
    pltpu.sync_copy(z_v, z_hbm.at[row, pl.ds(col, TPW)])


def _combine(slot, gmul, s1):
    mesh = plsc.VectorSubcoreMesh(core_axis_name="c", subcore_axis_name="s")
    return pl.kernel(
        _combine_body,
        out_type=jax.ShapeDtypeStruct((B_, S_), jnp.float32),
        mesh=mesh,
        scratch_types=[
            pltpu.VMEM((TPW,), jnp.int32),
            pltpu.VMEM((TPW,), jnp.float32),
            pltpu.VMEM((16,), jnp.float32),
            pltpu.VMEM((TPW,), jnp.float32),
            pltpu.SemaphoreType.DMA,
        ],
    )(slot, gmul, s1)


def _lsm_body(z_ref, out_ref):
    z = z_ref[...]
    m = jnp.max(z, axis=1, keepdims=True)
    out_ref[...] = z - m - jnp.log(jnp.sum(jnp.exp(z - m), axis=1, keepdims=True))


def _lsm(z2):
    return pl.pallas_call(
        _lsm_body,
        out_shape=jax.ShapeDtypeStruct((B_, S_), jnp.float32),
    )(z2)


def kernel(input, Wg, W1, b1, W2, b2):
    xf = input.reshape(T_, D_)
    w2sa = _w2sum_sc(W2.reshape(E_ * H_, D_))
    w2sb = _w2sum_tc(W2)
    slot, gmul, dst = _gating(xf, Wg)
    bufx = _scatter(dst, xf)
    s1 = _ffn(bufx, W1, b1.reshape(E_, 1, H_), w2sa, w2sb,
              b2.reshape(E_, 1, D_))
    z2 = _combine(slot, gmul, s1)
    return _lsm(z2)


# speedup vs baseline: 1.4312x; 1.4312x over previous
import functools

import jax
import jax.numpy as jnp
from jax import lax
from jax.experimental import pallas as pl
from jax.experimental.pallas import tpu as pltpu
from jax.experimental.pallas import tpu_sc as plsc

B_, S_, D_, H_, E_ = 2, 2048, 1024, 2048, 8
T_ = B_ * S_
C_ = 640
NSLOT = E_ * C_
NC, NS = 2, 16
NW = NC * NS
TPW = T_ // NW
RBUF = NSLOT + NW

TB = 512
HT = 512
HTW = 512
D2 = D_ // 2


def _w2sum_body(w2_ref, out_ref):
    out_ref[0] = jnp.sum(w2_ref[0], axis=1, keepdims=True)


def _w2sum(w2):
    return pl.pallas_call(
        _w2sum_body,
        grid=(E_, H_ // HTW),
        in_specs=[pl.BlockSpec((1, HTW, D_), lambda e, h: (e, h, 0))],
        out_specs=pl.BlockSpec((1, HTW, 1), lambda e, h: (e, h, 0)),
        out_shape=jax.ShapeDtypeStruct((E_, H_, 1), jnp.float32),
    )(w2)


def _gate_body(x_ref, wg_ref, slot_ref, gmul_ref, carry_ref):
    i = pl.program_id(0)
    x = x_ref[...]
    logits = jnp.dot(x, wg_ref[...], preferred_element_type=jnp.float32)
    lmax = jnp.max(logits, axis=1, keepdims=True)
    gval = 1.0 / jnp.sum(jnp.exp(logits - lmax), axis=1, keepdims=True)
    eids = lax.broadcasted_iota(jnp.int32, logits.shape, 1)
    eidx = jnp.min(jnp.where(logits >= lmax, eids, E_), axis=1, keepdims=True)

    @pl.when(i == 0)
    def _():
        carry_ref[...] = jnp.zeros_like(carry_ref)

    onehot = (eids == eidx).astype(jnp.float32)
    r = lax.broadcasted_iota(jnp.int32, (TB, TB), 0)
    c = lax.broadcasted_iota(jnp.int32, (TB, TB), 1)
    tri = (r >= c).astype(jnp.float32)
    cnt = jnp.dot(tri, onehot, preferred_element_type=jnp.float32)
    cnt = cnt + carry_ref[...]
    carry_ref[...] = cnt[TB - 1:TB, :]
    pos = jnp.sum(onehot * cnt, axis=1, keepdims=True).astype(jnp.int32) - 1
    valid = pos < C_
    slot = eidx * C_ + jnp.minimum(pos, C_ - 1)
    slot_ref[...] = slot.reshape(TB)
    gmul_ref[...] = jnp.where(valid, gval, 0.0).reshape(TB)


def _gating(xf, wg):
    return pl.pallas_call(
        _gate_body,
        grid=(T_ // TB,),
        in_specs=[
            pl.BlockSpec((TB, D_), lambda i: (i, 0)),
            pl.BlockSpec((D_, E_), lambda i: (0, 0)),
        ],
        out_specs=[
            pl.BlockSpec((TB,), lambda i: (i,)),
            pl.BlockSpec((TB,), lambda i: (i,)),
        ],
        out_shape=[
            jax.ShapeDtypeStruct((T_,), jnp.int32),
            jax.ShapeDtypeStruct((T_,), jnp.float32),
        ],
        scratch_shapes=[pltpu.VMEM((1, E_), jnp.float32)],
    )(xf, wg)


def _scatter_body(slot_hbm, gmul_hbm, x_hbm, bufx_hbm, sl_v, gm_v, rows_v, sem):
    wid = lax.axis_index("c") * NS + lax.axis_index("s")
    base = wid * TPW
    pltpu.sync_copy(slot_hbm.at[pl.ds(base, TPW)], sl_v)
    pltpu.sync_copy(gmul_hbm.at[pl.ds(base, TPW)], gm_v)
    for v in range(TPW // 16):
        sl = sl_v[pl.ds(v * 16, 16)]
        gm = gm_v[pl.ds(v * 16, 16)]
        idx = jnp.where(gm > 0.0, sl, NSLOT + wid)
        pltpu.sync_copy(x_hbm.at[pl.ds(base + v * 16, 16)], rows_v)
        pltpu.async_copy(rows_v, bufx_hbm.at[idx], sem).wait()


def _scatter(slot, gmul, xf):
    mesh = plsc.VectorSubcoreMesh(core_axis_name="c", subcore_axis_name="s")
    return pl.kernel(
        _scatter_body,
        out_type=jax.ShapeDtypeStruct((RBUF, D_), jnp.float32),
        mesh=mesh,
        scratch_types=[
            pltpu.VMEM((TPW,), jnp.int32),
            pltpu.VMEM((TPW,), jnp.float32),
            pltpu.VMEM((16, D_), jnp.float32),
            pltpu.SemaphoreType.DMA,
        ],
    )(slot, gmul, xf)


def _ffn_body(bufx_ref, w1a_ref, w1b_ref, b1_ref, w2s_ref, b2_ref,
              out_ref, acc_ref, xb_ref):
    e = pl.program_id(0)
    ht = pl.program_id(1)

    @pl.when(ht == 0)
    def _():
        xb_ref[...] = bufx_ref[...].astype(jnp.bfloat16)

    w1a = w1a_ref[0].astype(jnp.bfloat16)
    w1b = w1b_ref[0].astype(jnp.bfloat16)
    hblk = (jnp.dot(xb_ref[:, :D2], w1a, preferred_element_type=jnp.float32)
            + jnp.dot(xb_ref[:, D2:], w1b, preferred_element_type=jnp.float32))
    hblk = jnp.maximum(hblk + b1_ref[0], 0.0)
    part = jnp.dot(hblk, w2s_ref[0], preferred_element_type=jnp.float32)

    @pl.when(ht == 0)
    def _():
        acc_ref[...] = part + jnp.sum(b2_ref[0])

    @pl.when(ht != 0)
    def _():
        acc_ref[...] += part

    @pl.when(ht == H_ // HT - 1)
    def _():
        out_ref[pl.ds(e * C_, C_)] = acc_ref[...].reshape(C_)


def _ffn(bufx, w1, b1r, w2s, b2r):
    return pl.pallas_call(
        _ffn_body,
        grid=(E_, H_ // HT),
        in_specs=[
            pl.BlockSpec((C_, D_), lambda e, h: (e, 0)),
            pl.BlockSpec((1, D2, HT), lambda e, h: (e, 0, h)),
            pl.BlockSpec((1, D2, HT), lambda e, h: (e, 1, h)),
            pl.BlockSpec((1, 1, HT), lambda e, h: (e, 0, h)),
            pl.BlockSpec((1, HT, 1), lambda e, h: (e, h, 0)),
            pl.BlockSpec((1, 1, D_), lambda e, h: (e, 0, 0)),
        ],
        out_specs=pl.BlockSpec((NSLOT,), lambda e, h: (0,)),
        out_shape=jax.ShapeDtypeStruct((NSLOT,), jnp.float32),
        scratch_shapes=[pltpu.VMEM((C_, 1), jnp.float32),
                        pltpu.VMEM((C_, D_), jnp.bfloat16)],
    )(bufx, w1, w1, b1r, w2s, b2r)


def _combine_body(slot_hbm, gmul_hbm, s1_hbm, z_hbm, sl_v, gm_v, val_v, z_v, sem):
    wid = lax.axis_index("c") * NS + lax.axis_index("s")
    base = wid * TPW
    pltpu.sync_copy(slot_hbm.at[pl.ds(base, TPW)], sl_v)
    pltpu.sync_copy(gmul_hbm.at[pl.ds(base, TPW)], gm_v)
    pltpu.async_copy(s1_hbm.at[sl_v], val_v, sem).wait()
    for v in range(TPW // 16):
        z_v[pl.ds(v * 16, 16)] = (val_v[pl.ds(v * 16, 16)]
                                  * gm_v[pl.ds(v * 16, 16)])
    row = wid // (S_ // TPW)
    col = (wid % (S_ // TPW)) * TPW
    pltpu.sync_copy(z_v, z_hbm.at[row, pl.ds(col, TPW)])


def _combine(slot, gmul, s1):
    mesh = plsc.VectorSubcoreMesh(core_axis_name="c", subcore_axis_name="s")
    return pl.kernel(
        _combine_body,
        out_type=jax.ShapeDtypeStruct((B_, S_), jnp.float32),
        mesh=mesh,
        scratch_types=[
            pltpu.VMEM((TPW,), jnp.int32),
            pltpu.VMEM((TPW,), jnp.float32),
            pltpu.VMEM((TPW,), jnp.float32),
            pltpu.VMEM((TPW,), jnp.float32),
            pltpu.SemaphoreType.DMA,
        ],
    )(slot, gmul, s1)


def _lsm_body(z_ref, out_ref):
    z = z_ref[...]
    m = jnp.max(z, axis=1, keepdims=True)
    out_ref[...] = z - m - jnp.log(jnp.sum(jnp.exp(z - m), axis=1, keepdims=True))


def _lsm(z2):
    return pl.pallas_call(
        _lsm_body,
        out_shape=jax.ShapeDtypeStruct((B_, S_), jnp.float32),
    )(z2)


def kernel(input, Wg, W1, b1, W2, b2):
    xf = input.reshape(T_, D_)
    w2s = _w2sum(W2)
    slot, gmul = _gating(xf, Wg)
    bufx = _scatter(slot, gmul, xf)
    s1 = _ffn(bufx, W1, b1.reshape(E_, 1, H_), w2s, b2.reshape(E_, 1, D_))
    z2 = _combine(slot, gmul, s1)
    return _lsm(z2)


# speedup vs baseline: 1.5499x; 1.0830x over previous
import functools

import jax
import jax.numpy as jnp
from jax import lax
from jax.experimental import pallas as pl
from jax.experimental.pallas import tpu as pltpu
from jax.experimental.pallas import tpu_sc as plsc

B_, S_, D_, H_, E_ = 2, 2048, 1024, 2048, 8
T_ = B_ * S_
C_ = 640
NSLOT = E_ * C_
NC, NS = 2, 16
NW = NC * NS
TPW = T_ // NW
RBUF = NSLOT + NW

TB = 512
HT = 1024
HTW = 512
D2 = D_ // 2


def _w2sum_body(w2a_ref, w2b_ref, out_ref):
    out_ref[0] = (jnp.sum(w2a_ref[0], axis=1, keepdims=True)
                  + jnp.sum(w2b_ref[0], axis=1, keepdims=True))


def _w2sum(w2):
    return pl.pallas_call(
        _w2sum_body,
        grid=(E_, H_ // HTW),
        in_specs=[pl.BlockSpec((1, HTW, D2), lambda e, h: (e, h, 0)),
                  pl.BlockSpec((1, HTW, D2), lambda e, h: (e, h, 1))],
        out_specs=pl.BlockSpec((1, HTW, 1), lambda e, h: (e, h, 0)),
        out_shape=jax.ShapeDtypeStruct((E_, H_, 1), jnp.float32),
    )(w2, w2)


def _gate_body(x_ref, wg_ref, slot_ref, gmul_ref, carry_ref, tri_ref):
    i = pl.program_id(0)
    x = x_ref[...]
    logits = jnp.dot(x, wg_ref[...], preferred_element_type=jnp.float32)
    lmax = jnp.max(logits, axis=1, keepdims=True)
    gval = 1.0 / jnp.sum(jnp.exp(logits - lmax), axis=1, keepdims=True)
    eids = lax.broadcasted_iota(jnp.int32, logits.shape, 1)
    eidx = jnp.min(jnp.where(logits >= lmax, eids, E_), axis=1, keepdims=True)

    @pl.when(i == 0)
    def _():
        carry_ref[...] = jnp.zeros_like(carry_ref)
        r = lax.broadcasted_iota(jnp.int32, (TB, TB), 0)
        c = lax.broadcasted_iota(jnp.int32, (TB, TB), 1)
        tri_ref[...] = (r >= c).astype(jnp.float32)

    onehot = (eids == eidx).astype(jnp.float32)
    cnt = jnp.dot(tri_ref[...], onehot, preferred_element_type=jnp.float32)
    cnt = cnt + carry_ref[...]
    carry_ref[...] = cnt[TB - 1:TB, :]
    pos = jnp.sum(onehot * cnt, axis=1, keepdims=True).astype(jnp.int32) - 1
    valid = pos < C_
    slot = eidx * C_ + jnp.minimum(pos, C_ - 1)
    slot_ref[...] = slot.reshape(TB)
    gmul_ref[...] = jnp.where(valid, gval, 0.0).reshape(TB)


def _gating(xf, wg):
    return pl.pallas_call(
        _gate_body,
        grid=(T_ // TB,),
        in_specs=[
            pl.BlockSpec((TB, D_), lambda i: (i, 0)),
            pl.BlockSpec((D_, E_), lambda i: (0, 0)),
        ],
        out_specs=[
            pl.BlockSpec((TB,), lambda i: (i,)),
            pl.BlockSpec((TB,), lambda i: (i,)),
        ],
        out_shape=[
            jax.ShapeDtypeStruct((T_,), jnp.int32),
            jax.ShapeDtypeStruct((T_,), jnp.float32),
        ],
        scratch_shapes=[pltpu.VMEM((1, E_), jnp.float32),
                        pltpu.VMEM((TB, TB), jnp.float32)],
    )(xf, wg)


def _scatter_body(slot_hbm, gmul_hbm, x_hbm, bufx_hbm, sl_v, gm_v, rows_v, sem):
    wid = lax.axis_index("c") * NS + lax.axis_index("s")
    base = wid * TPW
    pltpu.sync_copy(slot_hbm.at[pl.ds(base, TPW)], sl_v)
    pltpu.sync_copy(gmul_hbm.at[pl.ds(base, TPW)], gm_v)
    for v in range(TPW // 16):
        sl = sl_v[pl.ds(v * 16, 16)]
        gm = gm_v[pl.ds(v * 16, 16)]
        idx = jnp.where(gm > 0.0, sl, NSLOT + wid)
        pltpu.sync_copy(x_hbm.at[pl.ds(base + v * 16, 16)], rows_v)
        pltpu.async_copy(rows_v, bufx_hbm.at[idx], sem).wait()


def _scatter(slot, gmul, xf):
    mesh = plsc.VectorSubcoreMesh(core_axis_name="c", subcore_axis_name="s")
    return pl.kernel(
        _scatter_body,
        out_type=jax.ShapeDtypeStruct((RBUF, D_), jnp.float32),
        mesh=mesh,
        scratch_types=[
            pltpu.VMEM((TPW,), jnp.int32),
            pltpu.VMEM((TPW,), jnp.float32),
            pltpu.VMEM((16, D_), jnp.float32),
            pltpu.SemaphoreType.DMA,
        ],
    )(slot, gmul, xf)


def _ffn_body(bufx_ref, w1a_ref, w1b_ref, b1_ref, w2s_ref, b2_ref,
              out_ref, acc_ref, xb_ref):
    e = pl.program_id(0)
    ht = pl.program_id(1)

    @pl.when(ht == 0)
    def _():
        xb_ref[...] = bufx_ref[...].astype(jnp.bfloat16)

    w1a = w1a_ref[0].astype(jnp.bfloat16)
    w1b = w1b_ref[0].astype(jnp.bfloat16)
    hblk = (jnp.dot(xb_ref[:, :D2], w1a, preferred_element_type=jnp.float32)
            + jnp.dot(xb_ref[:, D2:], w1b, preferred_element_type=jnp.float32))
    hblk = jnp.maximum(hblk + b1_ref[0], 0.0)
    part = jnp.dot(hblk, w2s_ref[0], preferred_element_type=jnp.float32)

    @pl.when(ht == 0)
    def _():
        acc_ref[...] = part + jnp.sum(b2_ref[0])

    @pl.when(ht != 0)
    def _():
        acc_ref[...] += part

    @pl.when(ht == H_ // HT - 1)
    def _():
        out_ref[pl.ds(e * C_, C_)] = acc_ref[...].reshape(C_)


def _ffn(bufx, w1, b1r, w2s, b2r):
    return pl.pallas_call(
        _ffn_body,
        grid=(E_, H_ // HT),
        in_specs=[
            pl.BlockSpec((C_, D_), lambda e, h: (e, 0)),
            pl.BlockSpec((1, D2, HT), lambda e, h: (e, 0, h)),
            pl.BlockSpec((1, D2, HT), lambda e, h: (e, 1, h)),
            pl.BlockSpec((1, 1, HT), lambda e, h: (e, 0, h)),
            pl.BlockSpec((1, HT, 1), lambda e, h: (e, h, 0)),
            pl.BlockSpec((1, 1, D_), lambda e, h: (e, 0, 0)),
        ],
        out_specs=pl.BlockSpec((NSLOT,), lambda e, h: (0,)),
        out_shape=jax.ShapeDtypeStruct((NSLOT,), jnp.float32),
        scratch_shapes=[pltpu.VMEM((C_, 1), jnp.float32),
                        pltpu.VMEM((C_, D_), jnp.bfloat16)],
    )(bufx, w1, w1, b1r, w2s, b2r)


def _combine_body(slot_hbm, gmul_hbm, s1_hbm, z_hbm, sl_v, gm_v, val_v, z_v, sem):
    wid = lax.axis_index("c") * NS + lax.axis_index("s")
    base = wid * TPW
    pltpu.sync_copy(slot_hbm.at[pl.ds(base, TPW)], sl_v)
    pltpu.sync_copy(gmul_hbm.at[pl.ds(base, TPW)], gm_v)
    pltpu.async_copy(s1_hbm.at[sl_v], val_v, sem).wait()
    for v in range(TPW // 16):
        z_v[pl.ds(v * 16, 16)] = (val_v[pl.ds(v * 16, 16)]
                                  * gm_v[pl.ds(v * 16, 16)])
    row = wid // (S_ // TPW)
    col = (wid % (S_ // TPW)) * TPW
    pltpu.sync_copy(z_v, z_hbm.at[row, pl.ds(col, TPW)])


def _combine(slot, gmul, s1):
    mesh = plsc.VectorSubcoreMesh(core_axis_name="c", subcore_axis_name="s")
    return pl.kernel(
        _combine_body,
        out_type=jax.ShapeDtypeStruct((B_, S_), jnp.float32),
        mesh=mesh,
        scratch_types=[
            pltpu.VMEM((TPW,), jnp.int32),
            pltpu.VMEM((TPW,), jnp.float32),
            pltpu.VMEM((TPW,), jnp.float32),
            pltpu.VMEM((TPW,), jnp.float32),
            pltpu.SemaphoreType.DMA,
        ],
    )(slot, gmul, s1)


def _lsm_body(z_ref, out_ref):
    z = z_ref[...]
    m = jnp.max(z, axis=1, keepdims=True)
    out_ref[...] = z - m - jnp.log(jnp.sum(jnp.exp(z - m), axis=1, keepdims=True))


def _lsm(z2):
    return pl.pallas_call(
        _lsm_body,
        out_shape=jax.ShapeDtypeStruct((B_, S_), jnp.float32),
    )(z2)


def kernel(input, Wg, W1, b1, W2, b2):
    xf = input.reshape(T_, D_)
    w2s = _w2sum(W2)
    slot, gmul = _gating(xf, Wg)
    bufx = _scatter(slot, gmul, xf)
    s1 = _ffn(bufx, W1, b1.reshape(E_, 1, H_), w2s, b2.reshape(E_, 1, D_))
    z2 = _combine(slot, gmul, s1)
    return _lsm(z2)
